# baseline (speedup 1.0000x reference)
import dataclasses
import functools

import jax
import jax.numpy as jnp
from jax import lax
from jax.experimental import pallas as pl
from jax.experimental.pallas import tpu as pltpu
from jax.experimental.pallas import tpu_sc as plsc

HP = 128
L = 16
NC, NS = 2, 16
NW = NC * NS
C = 32
NBUF = 2
NIB = 2
TB = 4


def _body(num_rels, emb_hbm, wrel_hbm, sidx_hbm, ridx_hbm, oidx_hbm, out_hbm,
          rel_v, cib, rib, sbuf, rows,
          semidx, semrow, semout, table_sh):
    wid = lax.axis_index("s") * NC + lax.axis_index("c")
    per_w = out_hbm.shape[0] // NW
    nchunks = per_w // C
    base_w = wid * per_w

    @pl.when(lax.axis_index("s") == 0)
    def _stage():
        pltpu.sync_copy(emb_hbm, table_sh)

    pltpu.sync_copy(wrel_hbm, rel_v)
    plsc.subcore_barrier()

    def stage_idx(k, ib):
        off = base_w + k * C
        pltpu.async_copy(sidx_hbm.at[pl.ds(off, C)], cib[ib].at[pl.ds(0, C)],
                         semidx[ib])
        pltpu.async_copy(oidx_hbm.at[pl.ds(off, C)], cib[ib].at[pl.ds(C, C)],
                         semidx[ib])
        pltpu.async_copy(ridx_hbm.at[pl.ds(off, C)], rib[ib].at[pl.ds(0, C)],
                         semidx[ib])

    def drain_idx(ib):
        pltpu.make_async_copy(sidx_hbm.at[pl.ds(0, C)], cib[ib].at[pl.ds(0, C)],
                              semidx[ib]).wait()
        pltpu.make_async_copy(sidx_hbm.at[pl.ds(0, C)], cib[ib].at[pl.ds(C, C)],
                              semidx[ib]).wait()
        pltpu.make_async_copy(ridx_hbm.at[pl.ds(0, C)], rib[ib].at[pl.ds(0, C)],
                              semidx[ib]).wait()

        @pl.loop(0, C, step=L)
        def _rmod(i):
            rib[ib][pl.ds(i, L)] = lax.rem(rib[ib][pl.ds(i, L)], num_rels) * HP

    def fire(rb, ib):
        pltpu.async_copy(table_sh.at[cib[ib]], rows[rb], semrow[rb])

    def drain_rows(rb):
        pltpu.make_async_copy(table_sh.at[cib[0]], rows[rb], semrow[rb]).wait()

    t_iota = lax.iota(jnp.int32, L)

    def compute(k, rb, ib):
        rw = rows[rb]
        sb = sbuf[rb]
        rbv = rib[ib]

        @pl.when(k >= NBUF)
        def _():
            pltpu.make_async_copy(sb, out_hbm.at[pl.ds(0, C)], semout[rb]).wait()

        for g in range(C // L):

            def tstep(it, scorevec, g=g):
                tb = g * L + it * TB
                rmvec = rbv[pl.ds(tb, L)]
                scores = []
                for c in range(TB):
                    t = tb + c
                    sref, oref = rw.at[t], rw.at[C + t]
                    rmb = rmvec[c]
                    acc0 = jnp.zeros((L,), jnp.float32)
                    acc1 = jnp.zeros((L,), jnp.float32)
                    for q in range(HP // L):
                        sv = sref[pl.ds(q * L, L)]
                        ov = oref[pl.ds(q * L, L)]
                        rv = rel_v[pl.ds(rmb + q * L, L)]
                        sa, sb_ = plsc.unpack(plsc.bitcast(sv, jnp.bfloat16),
                                              format=plsc.PackFormat.INTERLEAVED,
                                              preferred_element_type=jnp.float32)
                        oa, ob = plsc.unpack(plsc.bitcast(ov, jnp.bfloat16),
                                             format=plsc.PackFormat.INTERLEAVED,
                                             preferred_element_type=jnp.float32)
                        ra, rb_ = plsc.unpack(plsc.bitcast(rv, jnp.bfloat16),
                                              format=plsc.PackFormat.INTERLEAVED,
                                              preferred_element_type=jnp.float32)
                        acc0 = acc0 + (sa * oa) * ra
                        acc1 = acc1 + (sb_ * ob) * rb_
                    scores.append(jnp.sum(acc0 + acc1))
                for c, s in enumerate(scores):
                    scorevec = jnp.where(t_iota == it * TB + c, s, scorevec)
                return scorevec

            scorevec = lax.fori_loop(0, L // TB, tstep,
                                     jnp.zeros((L,), jnp.float32))
            sb[pl.ds(g * L, L)] = scorevec

        pltpu.async_copy(sb, out_hbm.at[pl.ds(base_w + k * C, C)], semout[rb])

    stage_idx(0, 0)
    drain_idx(0)
    fire(0, 0)

    @pl.loop(0, nchunks, step=NBUF)
    def _chunk(j):
        for p in range(NBUF):
            k = j + p

            @pl.when(k + 1 < nchunks)
            def _(k=k, p=p):
                stage_idx(k + 1, (p + 1) % NIB)

            drain_rows(p % NBUF)
            compute(k, p % NBUF, p % NIB)

            @pl.when(k + 1 < nchunks)
            def _(k=k, p=p):
                drain_idx((p + 1) % NIB)
                fire((p + 1) % NBUF, (p + 1) % NIB)

    for rb in range(NBUF):
        pltpu.make_async_copy(sbuf[rb], out_hbm.at[pl.ds(0, C)], semout[rb]).wait()


@functools.partial(jax.jit, static_argnames=("num_rels", "padded_b"))
def _score(emb_packed, wrel_flat, sidx, ridx, oidx, *, num_rels, padded_b):
    mesh = plsc.VectorSubcoreMesh(core_axis_name="c", subcore_axis_name="s")
    cp = pltpu.CompilerParams()
    fields = pltpu.CompilerParams.__dataclass_fields__
    for name, val in (("needs_layout_passes", False),
                      ("use_tc_tiling_on_sc", False),
                      ("disable_bounds_checks", True)):
        if name in fields:
            cp = dataclasses.replace(cp, **{name: val})
    f = pl.kernel(
        functools.partial(_body, num_rels),
        out_type=jax.ShapeDtypeStruct((padded_b,), jnp.float32),
        mesh=mesh,
        scratch_types=[
            pltpu.VMEM((num_rels * HP,), jnp.int32),
            [pltpu.VMEM((2 * C,), jnp.int32)] * NIB,
            [pltpu.VMEM((C + L,), jnp.int32)] * NIB,
            [pltpu.VMEM((C,), jnp.float32)] * NBUF,
            [pltpu.VMEM((2 * C, HP), jnp.int32)] * NBUF,
            [pltpu.SemaphoreType.DMA] * NIB,
            [pltpu.SemaphoreType.DMA] * NBUF,
            [pltpu.SemaphoreType.DMA] * NBUF,
            pltpu.VMEM_SHARED(emb_packed.shape, jnp.int32),
        ],
        compiler_params=cp,
    )
    return f(emb_packed, wrel_flat, sidx, ridx, oidx)


def kernel(embedding, w_relation, triplets):
    b = triplets.shape[0]
    tile = NW * C * NIB
    padded_b = ((b + tile - 1) // tile) * tile
    sidx = triplets[:, 0]
    ridx = triplets[:, 1]
    oidx = triplets[:, 2]
    if padded_b != b:
        z = jnp.zeros((padded_b - b,), jnp.int32)
        sidx = jnp.concatenate([sidx, z])
        ridx = jnp.concatenate([ridx, z])
        oidx = jnp.concatenate([oidx, z])
    n, h = embedding.shape
    emb_packed = lax.bitcast_convert_type(
        embedding.astype(jnp.bfloat16).reshape(n, h // 2, 2), jnp.int32)
    wrel_flat = lax.bitcast_convert_type(
        w_relation.astype(jnp.bfloat16).reshape(-1, 2), jnp.int32)
    scores = _score(emb_packed, wrel_flat, sidx, ridx, oidx,
                    num_rels=w_relation.shape[0], padded_b=padded_b)
    return scores[:b]


# speedup vs baseline: 2.2197x; 2.2197x over previous
import dataclasses
import functools

import jax
import jax.numpy as jnp
from jax import lax
from jax.experimental import pallas as pl
from jax.experimental.pallas import tpu as pltpu
from jax.experimental.pallas import tpu_sc as plsc

HP = 128
L = 16
NC, NS = 2, 16
NW = NC * NS
C = 32
NBUF = 2


def _body(num_rels, emb_hbm, wrel_hbm, sidx_hbm, ridx_hbm, oidx_hbm, out_hbm,
          sidx_v, oidx_v, rm_v, sbuf,
          srows, orows, rrows, sems, semo, semr, semout,
          table_sh, rel_sh):
    wid = lax.axis_index("s") * NC + lax.axis_index("c")
    per_w = out_hbm.shape[0] // NW
    nchunks = per_w // C
    base_w = wid * per_w

    @pl.when(lax.axis_index("s") == 0)
    def _stage():
        pltpu.sync_copy(emb_hbm, table_sh)
        pltpu.sync_copy(wrel_hbm, rel_sh)

    pltpu.sync_copy(sidx_hbm.at[pl.ds(base_w, per_w)], sidx_v)
    pltpu.sync_copy(oidx_hbm.at[pl.ds(base_w, per_w)], oidx_v)
    pltpu.sync_copy(ridx_hbm.at[pl.ds(base_w, per_w)], rm_v)

    @pl.loop(0, per_w, step=L)
    def _rmod(i):
        rm_v[pl.ds(i, L)] = lax.rem(rm_v[pl.ds(i, L)], num_rels)

    plsc.subcore_barrier()

    def fire(j, b):
        off = j * C
        pltpu.async_copy(table_sh.at[sidx_v.at[pl.ds(off, C)]], srows[b], sems[b])
        pltpu.async_copy(table_sh.at[oidx_v.at[pl.ds(off, C)]], orows[b], semo[b])
        pltpu.async_copy(rel_sh.at[rm_v.at[pl.ds(off, C)]], rrows[b], semr[b])

    def drain(b):
        pltpu.make_async_copy(table_sh.at[sidx_v.at[pl.ds(0, C)]], srows[b], sems[b]).wait()
        pltpu.make_async_copy(table_sh.at[oidx_v.at[pl.ds(0, C)]], orows[b], semo[b]).wait()
        pltpu.make_async_copy(rel_sh.at[rm_v.at[pl.ds(0, C)]], rrows[b], semr[b]).wait()

    t_iota = lax.iota(jnp.int32, L)

    def compute(j, b):
        sr, orr, rr = srows[b], orows[b], rrows[b]
        sb = sbuf[b]

        @pl.when(j >= NBUF)
        def _():
            pltpu.make_async_copy(sb, out_hbm.at[pl.ds(0, C)], semout[b]).wait()

        for g in range(C // L):

            def tstep(i, scorevec, g=g):
                t = g * L + i
                sref, oref, rref = sr.at[t], orr.at[t], rr.at[t]
                acc0 = jnp.zeros((L,), jnp.float32)
                acc1 = jnp.zeros((L,), jnp.float32)
                for k in range(HP // L):
                    sv = sref[pl.ds(k * L, L)]
                    ov = oref[pl.ds(k * L, L)]
                    rv = rref[pl.ds(k * L, L)]
                    sa, sb_ = plsc.unpack(plsc.bitcast(sv, jnp.bfloat16),
                                          format=plsc.PackFormat.INTERLEAVED,
                                          preferred_element_type=jnp.float32)
                    oa, ob = plsc.unpack(plsc.bitcast(ov, jnp.bfloat16),
                                         format=plsc.PackFormat.INTERLEAVED,
                                         preferred_element_type=jnp.float32)
                    ra, rb = plsc.unpack(plsc.bitcast(rv, jnp.bfloat16),
                                         format=plsc.PackFormat.INTERLEAVED,
                                         preferred_element_type=jnp.float32)
                    acc0 = acc0 + (sa * oa) * ra
                    acc1 = acc1 + (sb_ * ob) * rb
                s = jnp.sum(acc0 + acc1)
                return jnp.where(t_iota == i, s, scorevec)

            scorevec = lax.fori_loop(0, L, tstep, jnp.zeros((L,), jnp.float32))
            sb[pl.ds(g * L, L)] = scorevec

    for k in range(NBUF - 1):
        fire(k, k)

    @pl.loop(0, nchunks, step=NBUF)
    def _chunk(j):
        for b in range(NBUF):
            nxt = j + b + NBUF - 1

            @pl.when(nxt < nchunks)
            def _(nxt=nxt, b=b):
                fire(nxt, (b + NBUF - 1) % NBUF)

            drain(b)
            compute(j + b, b)
            pltpu.async_copy(sbuf[b], out_hbm.at[pl.ds(base_w + (j + b) * C, C)],
                             semout[b])

    for b in range(NBUF):
        pltpu.make_async_copy(sbuf[b], out_hbm.at[pl.ds(0, C)], semout[b]).wait()


@functools.partial(jax.jit, static_argnames=("num_rels", "padded_b"))
def _score(emb_packed, wrel_packed, sidx, ridx, oidx, *, num_rels, padded_b):
    mesh = plsc.VectorSubcoreMesh(core_axis_name="c", subcore_axis_name="s")
    cp = pltpu.CompilerParams()
    fields = pltpu.CompilerParams.__dataclass_fields__
    for name, val in (("needs_layout_passes", False),
                      ("use_tc_tiling_on_sc", False),
                      ("disable_bounds_checks", True)):
        if name in fields:
            cp = dataclasses.replace(cp, **{name: val})
    per_w = padded_b // NW
    f = pl.kernel(
        functools.partial(_body, num_rels),
        out_type=jax.ShapeDtypeStruct((padded_b,), jnp.float32),
        mesh=mesh,
        scratch_types=[
            pltpu.VMEM((per_w,), jnp.int32),
            pltpu.VMEM((per_w,), jnp.int32),
            pltpu.VMEM((per_w,), jnp.int32),
            [pltpu.VMEM((C,), jnp.float32)] * NBUF,
            [pltpu.VMEM((C, HP), jnp.int32)] * NBUF,
            [pltpu.VMEM((C, HP), jnp.int32)] * NBUF,
            [pltpu.VMEM((C, HP), jnp.int32)] * NBUF,
            [pltpu.SemaphoreType.DMA] * NBUF,
            [pltpu.SemaphoreType.DMA] * NBUF,
            [pltpu.SemaphoreType.DMA] * NBUF,
            [pltpu.SemaphoreType.DMA] * NBUF,
            pltpu.VMEM_SHARED(emb_packed.shape, jnp.int32),
            pltpu.VMEM_SHARED(wrel_packed.shape, jnp.int32),
        ],
        compiler_params=cp,
    )
    return f(emb_packed, wrel_packed, sidx, ridx, oidx)


def kernel(embedding, w_relation, triplets):
    b = triplets.shape[0]
    tile = NW * C * NBUF
    padded_b = ((b + tile - 1) // tile) * tile
    sidx = triplets[:, 0]
    ridx = triplets[:, 1]
    oidx = triplets[:, 2]
    if padded_b != b:
        z = jnp.zeros((padded_b - b,), jnp.int32)
        sidx = jnp.concatenate([sidx, z])
        ridx = jnp.concatenate([ridx, z])
        oidx = jnp.concatenate([oidx, z])
    n, h = embedding.shape
    emb_packed = lax.bitcast_convert_type(
        embedding.astype(jnp.bfloat16).reshape(n, h // 2, 2), jnp.int32)
    wrel_packed = lax.bitcast_convert_type(
        w_relation.astype(jnp.bfloat16).reshape(w_relation.shape[0], h // 2, 2),
        jnp.int32)
    scores = _score(emb_packed, wrel_packed, sidx, ridx, oidx,
                    num_rels=w_relation.shape[0], padded_b=padded_b)
    return scores[:b]
